# Initial kernel scaffold; baseline (speedup 1.0000x reference)
import jax
import jax.numpy as jnp
from jax.experimental import pallas as pl


def kernel(x, edge_index, edge_attr, eps, anet_w, anet_b, mlp_w, mlp_b):
    raise NotImplementedError("write your pallas kernel here")


# speedup vs baseline: 1.6339x; 1.6339x over previous
import functools

import jax
import jax.numpy as jnp
from jax import lax
from jax.experimental import pallas as pl
from jax.experimental.pallas import tpu as pltpu
from jax.experimental.pallas import tpu_sc as plsc

N = 10000
E = 320000
D = 128
DE = 16

DH = D // 2
NC = 2
NS = 16
L = 16

EPT = E // NS
CH = 128
NFULL = EPT // CH
REM = EPT - NFULL * CH
STG = 640
STRIDE = 624

EB = 2000
NB = 1000


def _ea_body(edge_attr_ref, wt_ref, b_ref, out_ref):
    ea = jnp.dot(edge_attr_ref[...], wt_ref[...],
                 preferred_element_type=jnp.float32) + b_ref[...]
    out_ref[0] = ea[:, :DH]
    out_ref[1] = ea[:, DH:]


def _edge_mlp(edge_attr, anet_wt, anet_b2):
    return pl.pallas_call(
        _ea_body,
        grid=(E // EB,),
        in_specs=[
            pl.BlockSpec((EB, DE), lambda i: (i, 0)),
            pl.BlockSpec((DE, D), lambda i: (0, 0)),
            pl.BlockSpec((1, D), lambda i: (0, 0)),
        ],
        out_specs=pl.BlockSpec((NC, EB, DH), lambda i: (0, i, 0)),
        out_shape=jax.ShapeDtypeStruct((NC, E, DH), jnp.float32),
    )(edge_attr, anet_wt, anet_b2)


def _sc_body(xs_hbm, src_hbm, dst_hbm, ea_hbm, out_hbm,
             x_sp, agg_sp, src_v, dst_v, ea_v, xr_v,
             srcr_v, dstr_v, ear_v, xrr_v, sem):
    c = lax.axis_index("c")
    s = lax.axis_index("s")

    def zrow(i, carry):
        for j in range(DH // L):
            ea_v[i, pl.ds(j * L, L)] = jnp.zeros((L,), jnp.float32)
        return carry
    lax.fori_loop(0, CH, zrow, 0)
    for k in range(STG // CH):
        pltpu.sync_copy(ea_v, agg_sp.at[pl.ds(s * STRIDE + k * CH, CH)])
    pltpu.sync_copy(xs_hbm.at[c, pl.ds(s * STRIDE, STG)],
                    x_sp.at[pl.ds(s * STRIDE, STG)])
    plsc.subcore_barrier()

    tile_base = s * EPT

    def do_chunk(base, src_r, dst_r, ea_r, xr_r, n):
        pltpu.sync_copy(src_hbm.at[pl.ds(base, n)], src_r)
        pltpu.sync_copy(dst_hbm.at[pl.ds(base, n)], dst_r)
        pltpu.sync_copy(ea_hbm.at[c, pl.ds(base, n)], ea_r)
        pltpu.async_copy(x_sp.at[src_r], xr_r, sem).wait()

        def edge(k, carry):
            for j in range(DH // L):
                sl = pl.ds(j * L, L)
                xr_r[k, sl] = jnp.maximum(xr_r[k, sl] + ea_r[k, sl], 0.0)
            return carry
        lax.fori_loop(0, n, edge, 0)
        pltpu.sync_copy(xr_r, agg_sp.at[dst_r], add=True)

    def chunk(i, carry):
        do_chunk(tile_base + i * CH, src_v, dst_v, ea_v, xr_v, CH)
        return carry
    lax.fori_loop(0, NFULL, chunk, 0)
    do_chunk(tile_base + NFULL * CH, srcr_v, dstr_v, ear_v, xrr_v, REM)

    plsc.subcore_barrier()
    pltpu.sync_copy(agg_sp.at[pl.ds(s * STRIDE, STG)],
                    out_hbm.at[c, pl.ds(s * STRIDE, STG)])


_sc_kernel = functools.partial(
    pl.kernel,
    out_type=jax.ShapeDtypeStruct((NC, N, DH), jnp.float32),
    mesh=plsc.VectorSubcoreMesh(
        core_axis_name="c", subcore_axis_name="s",
        num_cores=NC, num_subcores=NS),
    scratch_types=[
        pltpu.VMEM_SHARED((N, DH), jnp.float32),
        pltpu.VMEM_SHARED((N, DH), jnp.float32),
        pltpu.VMEM((CH,), jnp.int32),
        pltpu.VMEM((CH,), jnp.int32),
        pltpu.VMEM((CH, DH), jnp.float32),
        pltpu.VMEM((CH, DH), jnp.float32),
        pltpu.VMEM((REM,), jnp.int32),
        pltpu.VMEM((REM,), jnp.int32),
        pltpu.VMEM((REM, DH), jnp.float32),
        pltpu.VMEM((REM, DH), jnp.float32),
        pltpu.SemaphoreType.DMA,
    ],
    compiler_params=pltpu.CompilerParams(use_tc_tiling_on_sc=False),
)(_sc_body)


def _mlp_body(agg_ref, x_ref, wt_ref, b_ref, eps_ref, out_ref):
    h = jnp.dot(agg_ref[0], wt_ref[:DH], preferred_element_type=jnp.float32)
    h += jnp.dot(agg_ref[1], wt_ref[DH:D], preferred_element_type=jnp.float32)
    h += (1.0 + eps_ref[0]) * jnp.dot(x_ref[...], wt_ref[D:],
                                      preferred_element_type=jnp.float32)
    h += b_ref[...]
    h = jnp.where(h > 0, h, 0.2 * h)
    out_ref[...] = jnp.maximum(h, 0.0)


def _node_mlp(agg2, x, mlp_wt, mlp_b2, eps):
    return pl.pallas_call(
        _mlp_body,
        grid=(N // NB,),
        in_specs=[
            pl.BlockSpec((NC, NB, DH), lambda i: (0, i, 0)),
            pl.BlockSpec((NB, D), lambda i: (i, 0)),
            pl.BlockSpec((2 * D, D), lambda i: (0, 0)),
            pl.BlockSpec((1, D), lambda i: (0, 0)),
            pl.BlockSpec(memory_space=pltpu.SMEM),
        ],
        out_specs=pl.BlockSpec((NB, D), lambda i: (i, 0)),
        out_shape=jax.ShapeDtypeStruct((N, D), jnp.float32),
    )(agg2, x, mlp_wt, mlp_b2, eps)


def kernel(x, edge_index, edge_attr, eps, anet_w, anet_b, mlp_w, mlp_b):
    ea2 = _edge_mlp(edge_attr, anet_w.T, anet_b.reshape(1, D))
    xs = x.reshape(N, NC, DH).transpose(1, 0, 2)
    agg2 = _sc_kernel(xs, edge_index[0], edge_index[1], ea2)
    return _node_mlp(agg2, x, mlp_w.T, mlp_b.reshape(1, D), eps)
